# baseline (speedup 1.0000x reference)
import functools

import jax
import jax.numpy as jnp
from jax import lax
from jax.experimental import pallas as pl
from jax.experimental.pallas import tpu as pltpu
from jax.experimental.pallas import tpu_sc as plsc

_LANES = 16
_NBUF = 2


@functools.lru_cache(maxsize=None)
def _make_pack_kernel(V, D):
    assert D == 64 and V % 8 == 0, (V, D)
    info = plsc.get_sparse_core_info()
    NC, NS = info.num_cores, info.num_subcores
    NW = NC * NS
    nblk = V // 128
    tail = V - nblk * 128
    assert tail % 8 == 0
    mesh = plsc.VectorSubcoreMesh(core_axis_name="c", subcore_axis_name="s")

    @functools.partial(
        pl.kernel,
        out_type=jax.ShapeDtypeStruct((V // 2, 2 * D), jnp.float32),
        mesh=mesh,
        scratch_types=[
            pltpu.VMEM((_NBUF, 8, 8, 128), jnp.float32),
            pltpu.VMEM((_NBUF, 8, 8, 128), jnp.float32),
        ] + [pltpu.SemaphoreType.DMA] * (2 * _NBUF),
        compiler_params=pltpu.CompilerParams(
            use_tc_tiling_on_sc=True, needs_layout_passes=False),
        name="sc_table_pack",
    )
    def pack_kernel(tt_hbm, tail_hbm, out_hbm, a_v, o_v, *sems):
        sem_a = sems[:_NBUF]
        sem_o = sems[_NBUF:]
        wid = lax.axis_index("s") * NC + lax.axis_index("c")
        n_w = jnp.where(wid < (nblk % NW), nblk // NW + 1, nblk // NW)

        iota = lax.iota(jnp.int32, _LANES)
        ci = iota & 7
        cbs = [(d * _LANES + iota) >> 3 for d in range(D // _LANES)]

        def in_descs(blk, k):
            return [
                pltpu.make_async_copy(
                    tt_hbm.at[pl.ds(cb * 8, 8), pl.ds(blk * 128, 128)],
                    a_v.at[k, cb], sem_a[k])
                for cb in range(8)
            ]

        def out_descs(blk, k):
            return [
                pltpu.make_async_copy(
                    o_v.at[k, q], out_hbm.at[pl.ds(blk * 64 + q * 8, 8)],
                    sem_o[k])
                for q in range(8)
            ]

        for k in range(_NBUF):
            for d_ in in_descs(wid + k * NW, k):
                d_.start()

        def transpose_slab(k, jmax):
            def trow(i2, carry):
                for p in range(2):
                    j16 = jnp.full((_LANES,), 2 * i2 + p, jnp.int32)
                    for d in range(D // _LANES):
                        vec = plsc.load_gather(a_v.at[k], [cbs[d], ci, j16])
                        o_v[k, lax.shift_right_logical(i2, 3), i2 & 7,
                            pl.ds(p * D + d * _LANES, _LANES)] = vec
                return carry
            lax.fori_loop(0, jmax // 2, trow, 0)

        def step(i, k):
            @pl.when(i < n_w)
            def _():
                blk = wid + i * NW
                for d_ in in_descs(blk, k):
                    d_.wait()

                @pl.when(i >= _NBUF)
                def _():
                    for d_ in out_descs(blk, k):
                        d_.wait()

                transpose_slab(k, 128)
                for d_ in out_descs(blk, k):
                    d_.start()

                @pl.when(i + _NBUF < n_w)
                def _():
                    for d_ in in_descs(wid + (i + _NBUF) * NW, k):
                        d_.start()

        def body(g, carry):
            for k in range(_NBUF):
                step(g * _NBUF + k, k)
            return carry

        lax.fori_loop(0, (nblk // NW + _NBUF) // _NBUF, body, 0)
        for k in range(_NBUF):
            @pl.when(n_w >= _NBUF + k)
            def _():
                for d_ in out_descs(0, k):
                    d_.wait()

        if tail:
            @pl.when(wid == 0)
            def _():
                for cb in range(8):
                    pltpu.sync_copy(tail_hbm.at[pl.ds(cb * 8, 8)],
                                    a_v.at[0, cb])
                transpose_slab(0, tail)
                for q in range(tail // 16):
                    pltpu.sync_copy(
                        o_v.at[0, q],
                        out_hbm.at[pl.ds(nblk * 64 + q * 8, 8)])

    return pack_kernel


@functools.lru_cache(maxsize=None)
def _make_pool_kernel(V2, D, B, H):
    D2 = 2 * D
    info = plsc.get_sparse_core_info()
    NC, NS = info.num_cores, info.num_subcores
    NW = NC * NS
    assert B % NW == 0, (B, NW)
    assert D % _LANES == 0, D
    assert H % 8 == 0, H
    b_per_w = B // NW
    n_idx = b_per_w * H
    chunks = []
    off = 0
    while off < H:
        ln = min(128, H - off)
        chunks.append((off, ln))
        off += ln

    mesh = plsc.VectorSubcoreMesh(core_axis_name="c", subcore_axis_name="s")

    @functools.partial(
        pl.kernel,
        out_type=jax.ShapeDtypeStruct((B, D), jnp.float32),
        mesh=mesh,
        scratch_types=[
            pltpu.VMEM((n_idx,), jnp.int32),
            pltpu.VMEM((n_idx + _LANES,), jnp.int32),
            pltpu.VMEM((_NBUF, H, D2), jnp.float32),
            pltpu.VMEM((b_per_w, D), jnp.float32),
        ] + [pltpu.SemaphoreType.DMA] * _NBUF,
        compiler_params=pltpu.CompilerParams(use_tc_tiling_on_sc=True),
        name="sc_embed_sum_pool",
    )
    def pool_kernel(x_hbm, table_hbm, out_hbm, idx_v, off_v, rows_v, pooled_v,
                    *sems):
        wid = lax.axis_index("s") * NC + lax.axis_index("c")
        base = wid * b_per_w
        pltpu.sync_copy(x_hbm.at[pl.ds(base * H, n_idx)], idx_v)

        def prep(g, carry):
            v = idx_v[pl.ds(g * _LANES, _LANES)]
            off_v[pl.ds(g * _LANES, _LANES)] = (v & 1) * D
            idx_v[pl.ds(g * _LANES, _LANES)] = lax.shift_right_logical(v, 1)
            return carry

        lax.fori_loop(0, n_idx // _LANES, prep, 0, unroll=4)

        def gather_descs(e, k):
            return [
                pltpu.make_async_copy(
                    table_hbm.at[idx_v.at[pl.ds(e * H + off, ln)]],
                    rows_v.at[k].at[pl.ds(off, ln)],
                    sems[k],
                )
                for off, ln in chunks
            ]

        for k in range(_NBUF):
            for d_ in gather_descs(k, k):
                d_.start()

        def do_elem(e, k):
            for d_ in gather_descs(e, k):
                d_.wait()

            def inner(j, accs):
                o = off_v[pl.ds(e * H + j, _LANES)][0]
                return tuple(
                    accs[d] + rows_v[k, j, pl.ds(o + d * _LANES, _LANES)]
                    for d in range(D // _LANES)
                )

            zeros = tuple(
                jnp.zeros((_LANES,), jnp.float32) for _ in range(D // _LANES)
            )
            accs = lax.fori_loop(0, H, inner, zeros, unroll=4)
            for d in range(D // _LANES):
                pooled_v[e, pl.ds(d * _LANES, _LANES)] = accs[d]

            @pl.when(e + _NBUF < b_per_w)
            def _():
                for d_ in gather_descs(e + _NBUF, k):
                    d_.start()

        def body(i, carry):
            for k in range(_NBUF):
                do_elem(i * _NBUF + k, k)
            return carry

        lax.fori_loop(0, b_per_w // _NBUF, body, 0)
        pltpu.sync_copy(pooled_v, out_hbm.at[pl.ds(base, b_per_w)])

    return pool_kernel


def _head_body(p_ref, w_ref, b_ref, o_ref):
    o_ref[...] = (
        lax.dot_general(
            p_ref[...], w_ref[...],
            dimension_numbers=(((1,), (1,)), ((), ())),
            preferred_element_type=jnp.float32,
        )
        + b_ref[...]
    )


@functools.lru_cache(maxsize=None)
def _make_head_kernel(B, D, C):
    return pl.pallas_call(
        _head_body,
        out_shape=jax.ShapeDtypeStruct((B, C), jnp.float32),
    )


def kernel(x, table, W, b):
    B, H = x.shape
    V, D = table.shape
    C = W.shape[0]
    x_flat = x.reshape(B * H).astype(jnp.int32)
    tail = V % 128
    tail_t = jnp.pad(table[V - tail:].T, ((0, 0), (0, 128 - tail)))
    packed = _make_pack_kernel(V, D)(table.T, tail_t)
    pooled = _make_pool_kernel(V // 2, D, B, H)(x_flat, packed)
    return _make_head_kernel(B, D, C)(pooled, W, b.reshape(1, C))


# speedup vs baseline: 4.0139x; 4.0139x over previous
import functools

import jax
import jax.numpy as jnp
from jax import lax
from jax.experimental import pallas as pl
from jax.experimental.pallas import tpu as pltpu
from jax.experimental.pallas import tpu_sc as plsc

_LANES = 16
_NBUF = 2


@functools.lru_cache(maxsize=None)
def _make_pack_kernel(V, D):
    assert D == 64 and V % 8 == 0, (V, D)
    info = plsc.get_sparse_core_info()
    NC, NS = info.num_cores, info.num_subcores
    NW = NC * NS
    nblk = V // 128
    tail = V - nblk * 128
    assert tail % 8 == 0
    mesh = plsc.VectorSubcoreMesh(core_axis_name="c", subcore_axis_name="s")

    @functools.partial(
        pl.kernel,
        out_type=jax.ShapeDtypeStruct((V // 2, 2 * D), jnp.float32),
        mesh=mesh,
        scratch_types=[
            pltpu.VMEM((_NBUF, 8, 8, 128), jnp.float32),
            pltpu.VMEM((_NBUF, 8, 8, 128), jnp.float32),
        ] + [pltpu.SemaphoreType.DMA] * (2 * _NBUF),
        compiler_params=pltpu.CompilerParams(
            use_tc_tiling_on_sc=True, needs_layout_passes=False),
        name="sc_table_pack",
    )
    def pack_kernel(tt_hbm, tail_hbm, out_hbm, a_v, o_v, *sems):
        sem_a = sems[:_NBUF]
        sem_o = sems[_NBUF:]
        wid = lax.axis_index("s") * NC + lax.axis_index("c")
        n_w = jnp.where(wid < (nblk % NW), nblk // NW + 1, nblk // NW)

        iota = lax.iota(jnp.int32, _LANES)
        ci = iota & 7
        cbs = [(d * _LANES + iota) >> 3 for d in range(D // _LANES)]

        def in_descs(blk, k):
            return [
                pltpu.make_async_copy(
                    tt_hbm.at[pl.ds(cb * 8, 8), pl.ds(blk * 128, 128)],
                    a_v.at[k, cb], sem_a[k])
                for cb in range(8)
            ]

        def out_descs(blk, k):
            return [
                pltpu.make_async_copy(
                    o_v.at[k, q], out_hbm.at[pl.ds(blk * 64 + q * 8, 8)],
                    sem_o[k])
                for q in range(8)
            ]

        for k in range(_NBUF):
            for d_ in in_descs(wid + k * NW, k):
                d_.start()

        def transpose_slab(k, jmax):
            @functools.partial(plsc.parallel_loop, 0, jmax // 2, unroll=8)
            def trow(i2):
                for p in range(2):
                    j16 = jnp.full((_LANES,), 2 * i2 + p, jnp.int32)
                    for d in range(D // _LANES):
                        vec = plsc.load_gather(a_v.at[k], [cbs[d], ci, j16])
                        o_v[k, lax.shift_right_logical(i2, 3), i2 & 7,
                            pl.ds(p * D + d * _LANES, _LANES)] = vec

        def step(i, k):
            @pl.when(i < n_w)
            def _():
                blk = wid + i * NW
                for d_ in in_descs(blk, k):
                    d_.wait()

                @pl.when(i >= _NBUF)
                def _():
                    for d_ in out_descs(blk, k):
                        d_.wait()

                transpose_slab(k, 128)
                for d_ in out_descs(blk, k):
                    d_.start()

                @pl.when(i + _NBUF < n_w)
                def _():
                    for d_ in in_descs(wid + (i + _NBUF) * NW, k):
                        d_.start()

        def body(g, carry):
            for k in range(_NBUF):
                step(g * _NBUF + k, k)
            return carry

        lax.fori_loop(0, (nblk // NW + _NBUF) // _NBUF, body, 0)
        for k in range(_NBUF):
            @pl.when(n_w >= _NBUF + k)
            def _():
                for d_ in out_descs(0, k):
                    d_.wait()

        if tail:
            @pl.when(wid == 0)
            def _():
                for cb in range(8):
                    pltpu.sync_copy(tail_hbm.at[pl.ds(cb * 8, 8)],
                                    a_v.at[0, cb])
                transpose_slab(0, tail)
                for q in range(tail // 16):
                    pltpu.sync_copy(
                        o_v.at[0, q],
                        out_hbm.at[pl.ds(nblk * 64 + q * 8, 8)])

    return pack_kernel


@functools.lru_cache(maxsize=None)
def _make_pool_kernel(V2, D, B, H):
    D2 = 2 * D
    info = plsc.get_sparse_core_info()
    NC, NS = info.num_cores, info.num_subcores
    NW = NC * NS
    assert B % NW == 0, (B, NW)
    assert D % _LANES == 0, D
    assert H % 8 == 0, H
    b_per_w = B // NW
    n_idx = b_per_w * H
    chunks = []
    off = 0
    while off < H:
        ln = min(128, H - off)
        chunks.append((off, ln))
        off += ln

    mesh = plsc.VectorSubcoreMesh(core_axis_name="c", subcore_axis_name="s")

    @functools.partial(
        pl.kernel,
        out_type=jax.ShapeDtypeStruct((B, D), jnp.float32),
        mesh=mesh,
        scratch_types=[
            pltpu.VMEM((n_idx,), jnp.int32),
            pltpu.VMEM((n_idx + _LANES,), jnp.int32),
            pltpu.VMEM((_NBUF, H, D2), jnp.float32),
            pltpu.VMEM((b_per_w, D), jnp.float32),
        ] + [pltpu.SemaphoreType.DMA] * _NBUF,
        compiler_params=pltpu.CompilerParams(use_tc_tiling_on_sc=True),
        name="sc_embed_sum_pool",
    )
    def pool_kernel(x_hbm, table_hbm, out_hbm, idx_v, off_v, rows_v, pooled_v,
                    *sems):
        wid = lax.axis_index("s") * NC + lax.axis_index("c")
        base = wid * b_per_w
        pltpu.sync_copy(x_hbm.at[pl.ds(base * H, n_idx)], idx_v)

        def prep(g, carry):
            v = idx_v[pl.ds(g * _LANES, _LANES)]
            off_v[pl.ds(g * _LANES, _LANES)] = (v & 1) * D
            idx_v[pl.ds(g * _LANES, _LANES)] = lax.shift_right_logical(v, 1)
            return carry

        lax.fori_loop(0, n_idx // _LANES, prep, 0, unroll=4)

        def gather_descs(e, k):
            return [
                pltpu.make_async_copy(
                    table_hbm.at[idx_v.at[pl.ds(e * H + off, ln)]],
                    rows_v.at[k].at[pl.ds(off, ln)],
                    sems[k],
                )
                for off, ln in chunks
            ]

        for k in range(_NBUF):
            for d_ in gather_descs(k, k):
                d_.start()

        def do_elem(e, k):
            for d_ in gather_descs(e, k):
                d_.wait()

            def inner(j, accs):
                o = off_v[pl.ds(e * H + j, _LANES)][0]
                return tuple(
                    accs[d] + rows_v[k, j, pl.ds(o + d * _LANES, _LANES)]
                    for d in range(D // _LANES)
                )

            zeros = tuple(
                jnp.zeros((_LANES,), jnp.float32) for _ in range(D // _LANES)
            )
            accs = lax.fori_loop(0, H, inner, zeros, unroll=4)
            for d in range(D // _LANES):
                pooled_v[e, pl.ds(d * _LANES, _LANES)] = accs[d]

            @pl.when(e + _NBUF < b_per_w)
            def _():
                for d_ in gather_descs(e + _NBUF, k):
                    d_.start()

        def body(i, carry):
            for k in range(_NBUF):
                do_elem(i * _NBUF + k, k)
            return carry

        lax.fori_loop(0, b_per_w // _NBUF, body, 0)
        pltpu.sync_copy(pooled_v, out_hbm.at[pl.ds(base, b_per_w)])

    return pool_kernel


def _head_body(p_ref, w_ref, b_ref, o_ref):
    o_ref[...] = (
        lax.dot_general(
            p_ref[...], w_ref[...],
            dimension_numbers=(((1,), (1,)), ((), ())),
            preferred_element_type=jnp.float32,
        )
        + b_ref[...]
    )


@functools.lru_cache(maxsize=None)
def _make_head_kernel(B, D, C):
    return pl.pallas_call(
        _head_body,
        out_shape=jax.ShapeDtypeStruct((B, C), jnp.float32),
    )


def kernel(x, table, W, b):
    B, H = x.shape
    V, D = table.shape
    C = W.shape[0]
    x_flat = x.reshape(B * H).astype(jnp.int32)
    tail = V % 128
    tail_t = jnp.pad(table[V - tail:].T, ((0, 0), (0, 128 - tail)))
    packed = _make_pack_kernel(V, D)(table.T, tail_t)
    pooled = _make_pool_kernel(V // 2, D, B, H)(x_flat, packed)
    return _make_head_kernel(B, D, C)(pooled, W, b.reshape(1, C))
